# Initial kernel scaffold; baseline (speedup 1.0000x reference)
import jax
import jax.numpy as jnp
from jax.experimental import pallas as pl


def kernel(x, edge_index, batch, Wp, bp, W1, b1, W2, b2, Ws, bs, prelu_w):
    raise NotImplementedError("write your pallas kernel here")


# speedup vs baseline: 1.0092x; 1.0092x over previous
import functools

import jax
import jax.numpy as jnp
from jax.experimental import pallas as pl
from jax.experimental.pallas import tpu as pltpu

N = 10000
E = 320000
D_IN = 128
DH = 300
DP = 320
DOUT = 1024
DEPTH = 5
NG = 128
NPAD = 10240
BM = 512
NBLK = NPAD // BM

_INTERPRET = False


def _proj_body(x_ref, wp_ref, bp_ref, o_ref):
    o_ref[...] = jax.nn.relu(
        jnp.dot(x_ref[...], wp_ref[...], preferred_element_type=jnp.float32)
        + bp_ref[...])


def _proj(xp, Wpp, bpp):
    return pl.pallas_call(
        _proj_body,
        grid=(NBLK,),
        in_specs=[
            pl.BlockSpec((BM, D_IN), lambda i: (i, 0)),
            pl.BlockSpec((D_IN, DP), lambda i: (0, 0)),
            pl.BlockSpec((1, DP), lambda i: (0, 0)),
        ],
        out_specs=pl.BlockSpec((BM, DP), lambda i: (i, 0)),
        out_shape=jax.ShapeDtypeStruct((NPAD, DP), jnp.float32),
        interpret=_INTERPRET,
    )(xp, Wpp, bpp)


def _mlp_body(h_ref, a_ref, w1_ref, b1_ref, w2_ref, b2_ref, o_ref):
    t = h_ref[...] + a_ref[...]
    t = jax.nn.relu(
        jnp.dot(t, w1_ref[...], preferred_element_type=jnp.float32) + b1_ref[...])
    t = jnp.dot(t, w2_ref[...], preferred_element_type=jnp.float32) + b2_ref[...]
    o_ref[...] = jax.nn.relu(t)


def _mlp(h, agg, W1i, b1i, W2i, b2i):
    return pl.pallas_call(
        _mlp_body,
        grid=(NBLK,),
        in_specs=[
            pl.BlockSpec((BM, DP), lambda i: (i, 0)),
            pl.BlockSpec((BM, DP), lambda i: (i, 0)),
            pl.BlockSpec((DP, DP), lambda i: (0, 0)),
            pl.BlockSpec((1, DP), lambda i: (0, 0)),
            pl.BlockSpec((DP, DP), lambda i: (0, 0)),
            pl.BlockSpec((1, DP), lambda i: (0, 0)),
        ],
        out_specs=pl.BlockSpec((BM, DP), lambda i: (i, 0)),
        out_shape=jax.ShapeDtypeStruct((NPAD, DP), jnp.float32),
        interpret=_INTERPRET,
    )(h, agg, W1i, b1i, W2i, b2i)


def _last_body(h_ref, a_ref, w1_ref, b1_ref, w2_ref, b2_ref, batch_ref,
               ws_ref, bs_ref, pw_ref, o_ref, acc_ref):
    i = pl.program_id(0)

    @pl.when(i == 0)
    def _():
        acc_ref[...] = jnp.zeros_like(acc_ref)

    t = h_ref[...] + a_ref[...]
    t = jax.nn.relu(
        jnp.dot(t, w1_ref[...], preferred_element_type=jnp.float32) + b1_ref[...])
    t = jnp.dot(t, w2_ref[...], preferred_element_type=jnp.float32) + b2_ref[...]
    gids = batch_ref[...]
    seg = jax.lax.broadcasted_iota(jnp.int32, (NG, BM), 0)
    onehot = (seg == gids).astype(jnp.float32)
    acc_ref[...] += jnp.dot(onehot, t, preferred_element_type=jnp.float32)

    @pl.when(i == NBLK - 1)
    def _():
        r = jnp.dot(acc_ref[...], ws_ref[...],
                    preferred_element_type=jnp.float32) + bs_ref[...]
        pw = pw_ref[0]
        o_ref[...] = jnp.where(r >= 0.0, r, pw * r)


def _last(h, agg, W1i, b1i, W2i, b2i, batch2d, Wsp, bs2d, pw):
    return pl.pallas_call(
        _last_body,
        grid=(NBLK,),
        in_specs=[
            pl.BlockSpec((BM, DP), lambda i: (i, 0)),
            pl.BlockSpec((BM, DP), lambda i: (i, 0)),
            pl.BlockSpec((DP, DP), lambda i: (0, 0)),
            pl.BlockSpec((1, DP), lambda i: (0, 0)),
            pl.BlockSpec((DP, DP), lambda i: (0, 0)),
            pl.BlockSpec((1, DP), lambda i: (0, 0)),
            pl.BlockSpec((1, BM), lambda i: (0, i)),
            pl.BlockSpec((DP, DOUT), lambda i: (0, 0)),
            pl.BlockSpec((1, DOUT), lambda i: (0, 0)),
            pl.BlockSpec(memory_space=pltpu.SMEM),
        ],
        out_specs=pl.BlockSpec((NG, DOUT), lambda i: (0, 0)),
        out_shape=jax.ShapeDtypeStruct((NG, DOUT), jnp.float32),
        scratch_shapes=[pltpu.VMEM((NG, DP), jnp.float32)],
        interpret=_INTERPRET,
    )(h, agg, W1i, b1i, W2i, b2i, batch2d, Wsp, bs2d, pw)


def kernel(x, edge_index, batch, Wp, bp, W1, b1, W2, b2, Ws, bs, prelu_w):
    src = edge_index[0]
    dst = edge_index[1]
    xp = jnp.pad(x, ((0, NPAD - N), (0, 0)))
    Wpp = jnp.pad(Wp, ((0, 0), (0, DP - DH)))
    bpp = jnp.pad(bp, ((0, DP - DH),)).reshape(1, DP)
    W1p = jnp.pad(W1, ((0, 0), (0, DP - DH), (0, DP - DH)))
    b1p = jnp.pad(b1, ((0, 0), (0, DP - DH))).reshape(DEPTH, 1, DP)
    W2p = jnp.pad(W2, ((0, 0), (0, DP - DH), (0, DP - DH)))
    b2p = jnp.pad(b2, ((0, 0), (0, DP - DH))).reshape(DEPTH, 1, DP)
    Wsp = jnp.pad(Ws, ((0, DP - DH), (0, 0)))
    bs2d = bs.reshape(1, DOUT)
    batch2d = jnp.pad(batch, (0, NPAD - N), constant_values=NG).reshape(1, NPAD)
    pw = prelu_w.reshape(1)

    h = _proj(xp, Wpp, bpp)
    for i in range(DEPTH):
        agg = jnp.zeros_like(h).at[dst].add(h[src])
        if i < DEPTH - 1:
            h = _mlp(h, agg, W1p[i], b1p[i], W2p[i], b2p[i])
        else:
            r = _last(h, agg, W1p[i], b1p[i], W2p[i], b2p[i],
                      batch2d, Wsp, bs2d, pw)
    return r


# speedup vs baseline: 3.0249x; 2.9973x over previous
import functools

import jax
import jax.numpy as jnp
from jax import lax
from jax.experimental import pallas as pl
from jax.experimental.pallas import tpu as pltpu
from jax.experimental.pallas import tpu_sc as plsc

N = 10000
E = 320000
D_IN = 128
DH = 300
DP = 320
DHALF = DP // 2
DOUT = 1024
DEPTH = 5
NG = 128
NPAD = 10240
BM = 512
NBLK = NPAD // BM

NS = 16
EPT = E // NS
CH = 80
NCHUNK = EPT // CH
RPT = NPAD // NS

_sc_mesh = plsc.VectorSubcoreMesh(core_axis_name="c", subcore_axis_name="s")


@functools.partial(
    pl.kernel,
    out_type=(jax.ShapeDtypeStruct((NPAD, DHALF), jnp.float32),
              jax.ShapeDtypeStruct((NPAD, DHALF), jnp.float32)),
    mesh=_sc_mesh,
    scratch_types=[
        pltpu.VMEM_SHARED((NPAD, DHALF), jnp.float32),
        pltpu.VMEM((CH, DHALF), jnp.float32),
        pltpu.VMEM((CH,), jnp.int32),
        pltpu.VMEM((CH,), jnp.int32),
        pltpu.SemaphoreType.DMA,
    ],
    compiler_params=pltpu.CompilerParams(use_tc_tiling_on_sc=False),
)
def _sc_agg(hl_hbm, hr_hbm, src_hbm, dst_hbm, zeros_hbm,
            outl_hbm, outr_hbm,
            acc_sh, rows_v, srcb, dstb, sem):
    c = lax.axis_index("c")
    s = lax.axis_index("s")

    pltpu.sync_copy(zeros_hbm, rows_v)
    for k in range(RPT // CH):
        pltpu.sync_copy(rows_v,
                        acc_sh.at[pl.ds(s * RPT + k * CH, CH), :])
    plsc.subcore_barrier()

    base = s * EPT

    def do_edges(h_hbm):
        def body(i, carry):
            off = pl.multiple_of(base + i * CH, 8)
            pltpu.sync_copy(src_hbm.at[pl.ds(off, CH)], srcb)
            pltpu.sync_copy(dst_hbm.at[pl.ds(off, CH)], dstb)
            pltpu.async_copy(h_hbm.at[srcb], rows_v, sem).wait()
            pltpu.sync_copy(rows_v, acc_sh.at[dstb], add=True)
            return carry
        lax.fori_loop(0, NCHUNK, body, 0)

    @pl.when(c == 0)
    def _():
        do_edges(hl_hbm)

    @pl.when(c == 1)
    def _():
        do_edges(hr_hbm)

    plsc.subcore_barrier()

    def dump(out_hbm):
        for k in range(RPT // CH):
            r0 = s * RPT + k * CH
            pltpu.sync_copy(acc_sh.at[pl.ds(r0, CH), :], rows_v)
            pltpu.sync_copy(rows_v, out_hbm.at[pl.ds(r0, CH), :])

    @pl.when(c == 0)
    def _():
        dump(outl_hbm)

    @pl.when(c == 1)
    def _():
        dump(outr_hbm)


def _proj_body(x_ref, wa_ref, wb_ref, ba_ref, bb_ref, ol_ref, or_ref):
    xv = x_ref[...]
    ol_ref[...] = jax.nn.relu(
        jnp.dot(xv, wa_ref[...], preferred_element_type=jnp.float32) + ba_ref[...])
    or_ref[...] = jax.nn.relu(
        jnp.dot(xv, wb_ref[...], preferred_element_type=jnp.float32) + bb_ref[...])


def _proj(xp, Wpa, Wpb, bpa, bpb):
    return pl.pallas_call(
        _proj_body,
        grid=(NBLK,),
        in_specs=[
            pl.BlockSpec((BM, D_IN), lambda i: (i, 0)),
            pl.BlockSpec((D_IN, DHALF), lambda i: (0, 0)),
            pl.BlockSpec((D_IN, DHALF), lambda i: (0, 0)),
            pl.BlockSpec((1, DHALF), lambda i: (0, 0)),
            pl.BlockSpec((1, DHALF), lambda i: (0, 0)),
        ],
        out_specs=[pl.BlockSpec((BM, DHALF), lambda i: (i, 0)),
                   pl.BlockSpec((BM, DHALF), lambda i: (i, 0))],
        out_shape=[jax.ShapeDtypeStruct((NPAD, DHALF), jnp.float32),
                   jax.ShapeDtypeStruct((NPAD, DHALF), jnp.float32)],
    )(xp, Wpa, Wpb, bpa, bpb)


def _mlp_body(hl_ref, hr_ref, al_ref, ar_ref, w1a_ref, w1b_ref, b1_ref,
              w2a_ref, w2b_ref, b2a_ref, b2b_ref, ol_ref, or_ref):
    ul = hl_ref[...] + al_ref[...]
    ur = hr_ref[...] + ar_ref[...]
    u = jax.nn.relu(
        jnp.dot(ul, w1a_ref[...], preferred_element_type=jnp.float32)
        + jnp.dot(ur, w1b_ref[...], preferred_element_type=jnp.float32)
        + b1_ref[...])
    ol_ref[...] = jax.nn.relu(
        jnp.dot(u, w2a_ref[...], preferred_element_type=jnp.float32) + b2a_ref[...])
    or_ref[...] = jax.nn.relu(
        jnp.dot(u, w2b_ref[...], preferred_element_type=jnp.float32) + b2b_ref[...])


def _mlp(hl, hr, al, ar, W1a, W1b, b1, W2a, W2b, b2a, b2b):
    return pl.pallas_call(
        _mlp_body,
        grid=(NBLK,),
        in_specs=[
            pl.BlockSpec((BM, DHALF), lambda i: (i, 0)),
            pl.BlockSpec((BM, DHALF), lambda i: (i, 0)),
            pl.BlockSpec((BM, DHALF), lambda i: (i, 0)),
            pl.BlockSpec((BM, DHALF), lambda i: (i, 0)),
            pl.BlockSpec((DHALF, DP), lambda i: (0, 0)),
            pl.BlockSpec((DHALF, DP), lambda i: (0, 0)),
            pl.BlockSpec((1, DP), lambda i: (0, 0)),
            pl.BlockSpec((DP, DHALF), lambda i: (0, 0)),
            pl.BlockSpec((DP, DHALF), lambda i: (0, 0)),
            pl.BlockSpec((1, DHALF), lambda i: (0, 0)),
            pl.BlockSpec((1, DHALF), lambda i: (0, 0)),
        ],
        out_specs=[pl.BlockSpec((BM, DHALF), lambda i: (i, 0)),
                   pl.BlockSpec((BM, DHALF), lambda i: (i, 0))],
        out_shape=[jax.ShapeDtypeStruct((NPAD, DHALF), jnp.float32),
                   jax.ShapeDtypeStruct((NPAD, DHALF), jnp.float32)],
    )(hl, hr, al, ar, W1a, W1b, b1, W2a, W2b, b2a, b2b)


def _last_body(hl_ref, hr_ref, al_ref, ar_ref, w1a_ref, w1b_ref, b1_ref,
               w2_ref, b2_ref, batch_ref, ws_ref, bs_ref, pw_ref,
               o_ref, acc_ref):
    i = pl.program_id(0)

    @pl.when(i == 0)
    def _():
        acc_ref[...] = jnp.zeros_like(acc_ref)

    ul = hl_ref[...] + al_ref[...]
    ur = hr_ref[...] + ar_ref[...]
    u = jax.nn.relu(
        jnp.dot(ul, w1a_ref[...], preferred_element_type=jnp.float32)
        + jnp.dot(ur, w1b_ref[...], preferred_element_type=jnp.float32)
        + b1_ref[...])
    t = jnp.dot(u, w2_ref[...], preferred_element_type=jnp.float32) + b2_ref[...]
    gids = batch_ref[...]
    seg = jax.lax.broadcasted_iota(jnp.int32, (NG, BM), 0)
    onehot = (seg == gids).astype(jnp.float32)
    acc_ref[...] += jnp.dot(onehot, t, preferred_element_type=jnp.float32)

    @pl.when(i == NBLK - 1)
    def _():
        r = jnp.dot(acc_ref[...], ws_ref[...],
                    preferred_element_type=jnp.float32) + bs_ref[...]
        pw = pw_ref[0]
        o_ref[...] = jnp.where(r >= 0.0, r, pw * r)


def _last(hl, hr, al, ar, W1a, W1b, b1, W2f, b2f, batch2d, Wsp, bs2d, pw):
    return pl.pallas_call(
        _last_body,
        grid=(NBLK,),
        in_specs=[
            pl.BlockSpec((BM, DHALF), lambda i: (i, 0)),
            pl.BlockSpec((BM, DHALF), lambda i: (i, 0)),
            pl.BlockSpec((BM, DHALF), lambda i: (i, 0)),
            pl.BlockSpec((BM, DHALF), lambda i: (i, 0)),
            pl.BlockSpec((DHALF, DP), lambda i: (0, 0)),
            pl.BlockSpec((DHALF, DP), lambda i: (0, 0)),
            pl.BlockSpec((1, DP), lambda i: (0, 0)),
            pl.BlockSpec((DP, DP), lambda i: (0, 0)),
            pl.BlockSpec((1, DP), lambda i: (0, 0)),
            pl.BlockSpec((1, BM), lambda i: (0, i)),
            pl.BlockSpec((DP, DOUT), lambda i: (0, 0)),
            pl.BlockSpec((1, DOUT), lambda i: (0, 0)),
            pl.BlockSpec(memory_space=pltpu.SMEM),
        ],
        out_specs=pl.BlockSpec((NG, DOUT), lambda i: (0, 0)),
        out_shape=jax.ShapeDtypeStruct((NG, DOUT), jnp.float32),
        scratch_shapes=[pltpu.VMEM((NG, DP), jnp.float32)],
    )(hl, hr, al, ar, W1a, W1b, b1, W2f, b2f, batch2d, Wsp, bs2d, pw)


def kernel(x, edge_index, batch, Wp, bp, W1, b1, W2, b2, Ws, bs, prelu_w):
    src = edge_index[0]
    dst = edge_index[1]
    xp = jnp.pad(x, ((0, NPAD - N), (0, 0)))
    Wpp = jnp.pad(Wp, ((0, 0), (0, DP - DH)))
    bpp = jnp.pad(bp, ((0, DP - DH),)).reshape(1, DP)
    W1p = jnp.pad(W1, ((0, 0), (0, DP - DH), (0, DP - DH)))
    b1p = jnp.pad(b1, ((0, 0), (0, DP - DH))).reshape(DEPTH, 1, DP)
    W2p = jnp.pad(W2, ((0, 0), (0, DP - DH), (0, DP - DH)))
    b2p = jnp.pad(b2, ((0, 0), (0, DP - DH))).reshape(DEPTH, 1, DP)
    Wsp = jnp.pad(Ws, ((0, DP - DH), (0, 0)))
    bs2d = bs.reshape(1, DOUT)
    batch2d = jnp.pad(batch, (0, NPAD - N), constant_values=NG).reshape(1, NPAD)
    pw = prelu_w.reshape(1)
    zeros_stage = jnp.zeros((CH, DHALF), jnp.float32)

    Wpa, Wpb = Wpp[:, :DHALF], Wpp[:, DHALF:]
    bpa, bpb = bpp[:, :DHALF], bpp[:, DHALF:]
    W1a, W1b = W1p[:, :DHALF, :], W1p[:, DHALF:, :]
    W2a, W2b = W2p[:, :, :DHALF], W2p[:, :, DHALF:]
    b2a, b2b = b2p[:, :, :DHALF], b2p[:, :, DHALF:]

    hl, hr = _proj(xp, Wpa, Wpb, bpa, bpb)
    for i in range(DEPTH):
        al, ar = _sc_agg(hl, hr, src, dst, zeros_stage)
        if i < DEPTH - 1:
            hl, hr = _mlp(hl, hr, al, ar, W1a[i], W1b[i], b1p[i],
                          W2a[i], W2b[i], b2a[i], b2b[i])
        else:
            r = _last(hl, hr, al, ar, W1a[i], W1b[i], b1p[i],
                      W2p[i], b2p[i], batch2d, Wsp, bs2d, pw)
    return r


# speedup vs baseline: 6.6407x; 2.1954x over previous
import functools

import jax
import jax.numpy as jnp
from jax import lax
from jax.experimental import pallas as pl
from jax.experimental.pallas import tpu as pltpu
from jax.experimental.pallas import tpu_sc as plsc

N = 10000
E = 320000
D_IN = 128
DH = 300
DP = 320
DHALF = DP // 2
DOUT = 1024
DEPTH = 5
NG = 128
NPAD = 10240
BM = 512
NBLK = NPAD // BM

NS = 16
EPT = E // NS
CH = 80
NCHUNK = EPT // CH
RPT = NPAD // NS

_sc_mesh = plsc.VectorSubcoreMesh(core_axis_name="c", subcore_axis_name="s")


@functools.partial(
    pl.kernel,
    out_type=(jax.ShapeDtypeStruct((NPAD, DHALF), jnp.float32),
              jax.ShapeDtypeStruct((NPAD, DHALF), jnp.float32)),
    mesh=_sc_mesh,
    scratch_types=[
        pltpu.VMEM_SHARED((NPAD, DHALF), jnp.float32),
        pltpu.VMEM((CH, DHALF), jnp.float32),
        pltpu.VMEM((CH, DHALF), jnp.float32),
        pltpu.VMEM((CH,), jnp.int32),
        pltpu.VMEM((CH,), jnp.int32),
        pltpu.VMEM((CH,), jnp.int32),
        pltpu.VMEM((CH,), jnp.int32),
        pltpu.VMEM((CH,), jnp.int32),
        pltpu.VMEM((CH,), jnp.int32),
        pltpu.VMEM((CH,), jnp.int32),
        pltpu.VMEM((CH,), jnp.int32),
        pltpu.SemaphoreType.DMA,
        pltpu.SemaphoreType.DMA,
        pltpu.SemaphoreType.DMA,
        pltpu.SemaphoreType.DMA,
        pltpu.SemaphoreType.DMA,
        pltpu.SemaphoreType.DMA,
        pltpu.SemaphoreType.DMA,
        pltpu.SemaphoreType.DMA,
        pltpu.SemaphoreType.DMA,
        pltpu.SemaphoreType.DMA,
        pltpu.SemaphoreType.DMA,
        pltpu.SemaphoreType.DMA,
    ],
    compiler_params=pltpu.CompilerParams(use_tc_tiling_on_sc=False),
)
def _sc_agg(hl_hbm, hr_hbm, src_hbm, dst_hbm, zeros_hbm,
            outl_hbm, outr_hbm,
            acc_sh, rows0, rows1, sb0, sb1, sb2, sb3, db0, db1, db2, db3,
            sg0, sg1, ss0, ss1, sia0, sia1, sia2, sia3,
            sib0, sib1, sib2, sib3):
    c = lax.axis_index("c")
    s = lax.axis_index("s")
    rows = [rows0, rows1]
    srcb = [sb0, sb1, sb2, sb3]
    dstb = [db0, db1, db2, db3]
    sem_g = [sg0, sg1]
    sem_s = [ss0, ss1]
    sem_ia = [sia0, sia1, sia2, sia3]
    sem_ib = [sib0, sib1, sib2, sib3]

    pltpu.sync_copy(zeros_hbm, rows0)
    for k in range(RPT // CH):
        pltpu.sync_copy(rows0,
                        acc_sh.at[pl.ds(s * RPT + k * CH, CH), :])
    plsc.subcore_barrier()

    base = s * EPT

    def do_edges(h_hbm):
        def idx_start(ci, st):
            off = pl.multiple_of(base + ci * CH, 8)
            pltpu.async_copy(src_hbm.at[pl.ds(off, CH)], srcb[st], sem_ia[st])
            pltpu.async_copy(dst_hbm.at[pl.ds(off, CH)], dstb[st], sem_ib[st])

        def idx_wait(ci, st):
            off = pl.multiple_of(base + ci * CH, 8)
            pltpu.make_async_copy(src_hbm.at[pl.ds(off, CH)], srcb[st],
                                  sem_ia[st]).wait()
            pltpu.make_async_copy(dst_hbm.at[pl.ds(off, CH)], dstb[st],
                                  sem_ib[st]).wait()

        def gather_start(st, rb):
            pltpu.async_copy(h_hbm.at[srcb[st]], rows[rb], sem_g[rb])

        def gather_wait(st, rb):
            pltpu.make_async_copy(h_hbm.at[srcb[st]], rows[rb],
                                  sem_g[rb]).wait()

        def scat_start(st, rb):
            pltpu.async_copy(rows[rb], acc_sh.at[dstb[st]], sem_s[rb],
                             add=True)

        def scat_wait(st, rb):
            pltpu.make_async_copy(rows[rb], acc_sh.at[dstb[st]],
                                  sem_s[rb]).wait()

        for j in range(4):
            idx_start(j, j)
        idx_wait(0, 0)
        gather_start(0, 0)
        idx_wait(1, 1)
        gather_start(1, 1)
        gather_wait(0, 0)
        scat_start(0, 0)

        def body(k, carry):
            for b in range(4):
                i = 2 + 4 * k + b
                rb = b & 1
                st = (2 + b) & 3
                rb_prev = (1 + b) & 1
                st_prev = (1 + b) & 3
                st_old = b & 3
                scat_wait(st_old, rb)
                idx_wait(i, st)
                gather_start(st, rb)
                gather_wait(st_prev, rb_prev)
                scat_start(st_prev, rb_prev)
                if b < 2:
                    idx_start(i + 2, st_old)
                else:
                    @pl.when(k < NCHUNK // 4 - 1)
                    def _():
                        idx_start(i + 2, st_old)
            return carry
        lax.fori_loop(0, (NCHUNK - 2) // 4, body, 0)

        last = NCHUNK - 1
        gather_wait(last & 3, last & 1)
        scat_start(last & 3, last & 1)
        scat_wait((last - 1) & 3, (last - 1) & 1)
        scat_wait(last & 3, last & 1)

    @pl.when(c == 0)
    def _():
        do_edges(hl_hbm)

    @pl.when(c == 1)
    def _():
        do_edges(hr_hbm)

    plsc.subcore_barrier()

    def dump(out_hbm):
        for k in range(RPT // CH):
            r0 = s * RPT + k * CH
            pltpu.sync_copy(acc_sh.at[pl.ds(r0, CH), :], rows0)
            pltpu.sync_copy(rows0, out_hbm.at[pl.ds(r0, CH), :])

    @pl.when(c == 0)
    def _():
        dump(outl_hbm)

    @pl.when(c == 1)
    def _():
        dump(outr_hbm)


def _proj_body(x_ref, wa_ref, wb_ref, ba_ref, bb_ref, ol_ref, or_ref):
    xv = x_ref[...]
    ol_ref[...] = jax.nn.relu(
        jnp.dot(xv, wa_ref[...], preferred_element_type=jnp.float32) + ba_ref[...])
    or_ref[...] = jax.nn.relu(
        jnp.dot(xv, wb_ref[...], preferred_element_type=jnp.float32) + bb_ref[...])


def _proj(xp, Wpa, Wpb, bpa, bpb):
    return pl.pallas_call(
        _proj_body,
        grid=(NBLK,),
        in_specs=[
            pl.BlockSpec((BM, D_IN), lambda i: (i, 0)),
            pl.BlockSpec((D_IN, DHALF), lambda i: (0, 0)),
            pl.BlockSpec((D_IN, DHALF), lambda i: (0, 0)),
            pl.BlockSpec((1, DHALF), lambda i: (0, 0)),
            pl.BlockSpec((1, DHALF), lambda i: (0, 0)),
        ],
        out_specs=[pl.BlockSpec((BM, DHALF), lambda i: (i, 0)),
                   pl.BlockSpec((BM, DHALF), lambda i: (i, 0))],
        out_shape=[jax.ShapeDtypeStruct((NPAD, DHALF), jnp.float32),
                   jax.ShapeDtypeStruct((NPAD, DHALF), jnp.float32)],
    )(xp, Wpa, Wpb, bpa, bpb)


def _mlp_body(hl_ref, hr_ref, al_ref, ar_ref, w1a_ref, w1b_ref, b1_ref,
              w2a_ref, w2b_ref, b2a_ref, b2b_ref, ol_ref, or_ref):
    ul = hl_ref[...] + al_ref[...]
    ur = hr_ref[...] + ar_ref[...]
    u = jax.nn.relu(
        jnp.dot(ul, w1a_ref[...], preferred_element_type=jnp.float32)
        + jnp.dot(ur, w1b_ref[...], preferred_element_type=jnp.float32)
        + b1_ref[...])
    ol_ref[...] = jax.nn.relu(
        jnp.dot(u, w2a_ref[...], preferred_element_type=jnp.float32) + b2a_ref[...])
    or_ref[...] = jax.nn.relu(
        jnp.dot(u, w2b_ref[...], preferred_element_type=jnp.float32) + b2b_ref[...])


def _mlp(hl, hr, al, ar, W1a, W1b, b1, W2a, W2b, b2a, b2b):
    return pl.pallas_call(
        _mlp_body,
        grid=(NBLK,),
        in_specs=[
            pl.BlockSpec((BM, DHALF), lambda i: (i, 0)),
            pl.BlockSpec((BM, DHALF), lambda i: (i, 0)),
            pl.BlockSpec((BM, DHALF), lambda i: (i, 0)),
            pl.BlockSpec((BM, DHALF), lambda i: (i, 0)),
            pl.BlockSpec((DHALF, DP), lambda i: (0, 0)),
            pl.BlockSpec((DHALF, DP), lambda i: (0, 0)),
            pl.BlockSpec((1, DP), lambda i: (0, 0)),
            pl.BlockSpec((DP, DHALF), lambda i: (0, 0)),
            pl.BlockSpec((DP, DHALF), lambda i: (0, 0)),
            pl.BlockSpec((1, DHALF), lambda i: (0, 0)),
            pl.BlockSpec((1, DHALF), lambda i: (0, 0)),
        ],
        out_specs=[pl.BlockSpec((BM, DHALF), lambda i: (i, 0)),
                   pl.BlockSpec((BM, DHALF), lambda i: (i, 0))],
        out_shape=[jax.ShapeDtypeStruct((NPAD, DHALF), jnp.float32),
                   jax.ShapeDtypeStruct((NPAD, DHALF), jnp.float32)],
    )(hl, hr, al, ar, W1a, W1b, b1, W2a, W2b, b2a, b2b)


def _last_body(hl_ref, hr_ref, al_ref, ar_ref, w1a_ref, w1b_ref, b1_ref,
               w2_ref, b2_ref, batch_ref, ws_ref, bs_ref, pw_ref,
               o_ref, acc_ref):
    i = pl.program_id(0)

    @pl.when(i == 0)
    def _():
        acc_ref[...] = jnp.zeros_like(acc_ref)

    ul = hl_ref[...] + al_ref[...]
    ur = hr_ref[...] + ar_ref[...]
    u = jax.nn.relu(
        jnp.dot(ul, w1a_ref[...], preferred_element_type=jnp.float32)
        + jnp.dot(ur, w1b_ref[...], preferred_element_type=jnp.float32)
        + b1_ref[...])
    t = jnp.dot(u, w2_ref[...], preferred_element_type=jnp.float32) + b2_ref[...]
    gids = batch_ref[...]
    seg = jax.lax.broadcasted_iota(jnp.int32, (NG, BM), 0)
    onehot = (seg == gids).astype(jnp.float32)
    acc_ref[...] += jnp.dot(onehot, t, preferred_element_type=jnp.float32)

    @pl.when(i == NBLK - 1)
    def _():
        r = jnp.dot(acc_ref[...], ws_ref[...],
                    preferred_element_type=jnp.float32) + bs_ref[...]
        pw = pw_ref[0]
        o_ref[...] = jnp.where(r >= 0.0, r, pw * r)


def _last(hl, hr, al, ar, W1a, W1b, b1, W2f, b2f, batch2d, Wsp, bs2d, pw):
    return pl.pallas_call(
        _last_body,
        grid=(NBLK,),
        in_specs=[
            pl.BlockSpec((BM, DHALF), lambda i: (i, 0)),
            pl.BlockSpec((BM, DHALF), lambda i: (i, 0)),
            pl.BlockSpec((BM, DHALF), lambda i: (i, 0)),
            pl.BlockSpec((BM, DHALF), lambda i: (i, 0)),
            pl.BlockSpec((DHALF, DP), lambda i: (0, 0)),
            pl.BlockSpec((DHALF, DP), lambda i: (0, 0)),
            pl.BlockSpec((1, DP), lambda i: (0, 0)),
            pl.BlockSpec((DP, DP), lambda i: (0, 0)),
            pl.BlockSpec((1, DP), lambda i: (0, 0)),
            pl.BlockSpec((1, BM), lambda i: (0, i)),
            pl.BlockSpec((DP, DOUT), lambda i: (0, 0)),
            pl.BlockSpec((1, DOUT), lambda i: (0, 0)),
            pl.BlockSpec(memory_space=pltpu.SMEM),
        ],
        out_specs=pl.BlockSpec((NG, DOUT), lambda i: (0, 0)),
        out_shape=jax.ShapeDtypeStruct((NG, DOUT), jnp.float32),
        scratch_shapes=[pltpu.VMEM((NG, DP), jnp.float32)],
    )(hl, hr, al, ar, W1a, W1b, b1, W2f, b2f, batch2d, Wsp, bs2d, pw)


def kernel(x, edge_index, batch, Wp, bp, W1, b1, W2, b2, Ws, bs, prelu_w):
    src = edge_index[0]
    dst = edge_index[1]
    xp = jnp.pad(x, ((0, NPAD - N), (0, 0)))
    Wpp = jnp.pad(Wp, ((0, 0), (0, DP - DH)))
    bpp = jnp.pad(bp, ((0, DP - DH),)).reshape(1, DP)
    W1p = jnp.pad(W1, ((0, 0), (0, DP - DH), (0, DP - DH)))
    b1p = jnp.pad(b1, ((0, 0), (0, DP - DH))).reshape(DEPTH, 1, DP)
    W2p = jnp.pad(W2, ((0, 0), (0, DP - DH), (0, DP - DH)))
    b2p = jnp.pad(b2, ((0, 0), (0, DP - DH))).reshape(DEPTH, 1, DP)
    Wsp = jnp.pad(Ws, ((0, DP - DH), (0, 0)))
    bs2d = bs.reshape(1, DOUT)
    batch2d = jnp.pad(batch, (0, NPAD - N), constant_values=NG).reshape(1, NPAD)
    pw = prelu_w.reshape(1)
    zeros_stage = jnp.zeros((CH, DHALF), jnp.float32)

    Wpa, Wpb = Wpp[:, :DHALF], Wpp[:, DHALF:]
    bpa, bpb = bpp[:, :DHALF], bpp[:, DHALF:]
    W1a, W1b = W1p[:, :DHALF, :], W1p[:, DHALF:, :]
    W2a, W2b = W2p[:, :, :DHALF], W2p[:, :, DHALF:]
    b2a, b2b = b2p[:, :, :DHALF], b2p[:, :, DHALF:]

    hl, hr = _proj(xp, Wpa, Wpb, bpa, bpb)
    for i in range(DEPTH):
        al, ar = _sc_agg(hl, hr, src, dst, zeros_stage)
        if i < DEPTH - 1:
            hl, hr = _mlp(hl, hr, al, ar, W1a[i], W1b[i], b1p[i],
                          W2a[i], W2b[i], b2a[i], b2b[i])
        else:
            r = _last(hl, hr, al, ar, W1a[i], W1b[i], b1p[i],
                      W2p[i], b2p[i], batch2d, Wsp, bs2d, pw)
    return r


# speedup vs baseline: 6.7177x; 1.0116x over previous
import functools

import jax
import jax.numpy as jnp
from jax import lax
from jax.experimental import pallas as pl
from jax.experimental.pallas import tpu as pltpu
from jax.experimental.pallas import tpu_sc as plsc

N = 10000
E = 320000
D_IN = 128
DH = 300
DP = 320
DHALF = DP // 2
DOUT = 1024
DEPTH = 5
NG = 128
NPAD = 10240
BM = 512
NBLK = NPAD // BM

NS = 16
EPT = E // NS
CH = 80
NCHUNK = EPT // CH
RPT = NPAD // NS

_sc_mesh = plsc.VectorSubcoreMesh(core_axis_name="c", subcore_axis_name="s")


@functools.partial(
    pl.kernel,
    out_type=(jax.ShapeDtypeStruct((NPAD, DHALF), jnp.float32),
              jax.ShapeDtypeStruct((NPAD, DHALF), jnp.float32)),
    mesh=_sc_mesh,
    scratch_types=[
        pltpu.VMEM_SHARED((NPAD, DHALF), jnp.float32),
        pltpu.VMEM((CH, DHALF), jnp.float32),
        pltpu.VMEM((CH, DHALF), jnp.float32),
        pltpu.VMEM((CH,), jnp.int32),
        pltpu.VMEM((CH,), jnp.int32),
        pltpu.VMEM((CH,), jnp.int32),
        pltpu.VMEM((CH,), jnp.int32),
        pltpu.VMEM((CH,), jnp.int32),
        pltpu.VMEM((CH,), jnp.int32),
        pltpu.VMEM((CH,), jnp.int32),
        pltpu.VMEM((CH,), jnp.int32),
        pltpu.SemaphoreType.DMA,
        pltpu.SemaphoreType.DMA,
        pltpu.SemaphoreType.DMA,
        pltpu.SemaphoreType.DMA,
        pltpu.SemaphoreType.DMA,
        pltpu.SemaphoreType.DMA,
        pltpu.SemaphoreType.DMA,
        pltpu.SemaphoreType.DMA,
        pltpu.SemaphoreType.DMA,
        pltpu.SemaphoreType.DMA,
        pltpu.SemaphoreType.DMA,
        pltpu.SemaphoreType.DMA,
    ],
    compiler_params=pltpu.CompilerParams(use_tc_tiling_on_sc=False),
)
def _sc_agg(hl_hbm, hr_hbm, src_hbm, dst_hbm,
            outl_hbm, outr_hbm,
            acc_sh, rows0, rows1, sb0, sb1, sb2, sb3, db0, db1, db2, db3,
            sg0, sg1, ss0, ss1, sia0, sia1, sia2, sia3,
            sib0, sib1, sib2, sib3):
    c = lax.axis_index("c")
    s = lax.axis_index("s")
    rows = [rows0, rows1]
    srcb = [sb0, sb1, sb2, sb3]
    dstb = [db0, db1, db2, db3]
    sem_g = [sg0, sg1]
    sem_s = [ss0, ss1]
    sem_ia = [sia0, sia1, sia2, sia3]
    sem_ib = [sib0, sib1, sib2, sib3]

    r0 = s * RPT

    @pl.when(c == 0)
    def _():
        pltpu.sync_copy(hl_hbm.at[pl.ds(r0, RPT), :],
                        acc_sh.at[pl.ds(r0, RPT), :])

    @pl.when(c == 1)
    def _():
        pltpu.sync_copy(hr_hbm.at[pl.ds(r0, RPT), :],
                        acc_sh.at[pl.ds(r0, RPT), :])

    plsc.subcore_barrier()

    base = s * EPT

    def do_edges(h_hbm):
        def idx_start(ci, st):
            off = pl.multiple_of(base + ci * CH, 8)
            pltpu.async_copy(src_hbm.at[pl.ds(off, CH)], srcb[st], sem_ia[st])
            pltpu.async_copy(dst_hbm.at[pl.ds(off, CH)], dstb[st], sem_ib[st])

        def idx_wait(ci, st):
            off = pl.multiple_of(base + ci * CH, 8)
            pltpu.make_async_copy(src_hbm.at[pl.ds(off, CH)], srcb[st],
                                  sem_ia[st]).wait()
            pltpu.make_async_copy(dst_hbm.at[pl.ds(off, CH)], dstb[st],
                                  sem_ib[st]).wait()

        def gather_start(st, rb):
            pltpu.async_copy(h_hbm.at[srcb[st]], rows[rb], sem_g[rb])

        def gather_wait(st, rb):
            pltpu.make_async_copy(h_hbm.at[srcb[st]], rows[rb],
                                  sem_g[rb]).wait()

        def scat_start(st, rb):
            pltpu.async_copy(rows[rb], acc_sh.at[dstb[st]], sem_s[rb],
                             add=True)

        def scat_wait(st, rb):
            pltpu.make_async_copy(rows[rb], acc_sh.at[dstb[st]],
                                  sem_s[rb]).wait()

        for j in range(4):
            idx_start(j, j)
        idx_wait(0, 0)
        gather_start(0, 0)
        idx_wait(1, 1)
        gather_start(1, 1)
        gather_wait(0, 0)
        scat_start(0, 0)

        def body(k, carry):
            for b in range(4):
                i = 2 + 4 * k + b
                rb = b & 1
                st = (2 + b) & 3
                rb_prev = (1 + b) & 1
                st_prev = (1 + b) & 3
                st_old = b & 3
                scat_wait(st_old, rb)
                idx_wait(i, st)
                gather_start(st, rb)
                gather_wait(st_prev, rb_prev)
                scat_start(st_prev, rb_prev)
                if b < 2:
                    idx_start(i + 2, st_old)
                else:
                    @pl.when(k < NCHUNK // 4 - 1)
                    def _():
                        idx_start(i + 2, st_old)
            return carry
        lax.fori_loop(0, (NCHUNK - 2) // 4, body, 0)

        last = NCHUNK - 1
        gather_wait(last & 3, last & 1)
        scat_start(last & 3, last & 1)
        scat_wait((last - 1) & 3, (last - 1) & 1)
        scat_wait(last & 3, last & 1)

    @pl.when(c == 0)
    def _():
        do_edges(hl_hbm)

    @pl.when(c == 1)
    def _():
        do_edges(hr_hbm)

    plsc.subcore_barrier()

    def dump(out_hbm):
        pltpu.sync_copy(acc_sh.at[pl.ds(r0, RPT), :],
                        out_hbm.at[pl.ds(r0, RPT), :])

    @pl.when(c == 0)
    def _():
        dump(outl_hbm)

    @pl.when(c == 1)
    def _():
        dump(outr_hbm)


def _proj_body(x_ref, wa_ref, wb_ref, ba_ref, bb_ref, ol_ref, or_ref):
    xv = x_ref[...]
    ol_ref[...] = jax.nn.relu(
        jnp.dot(xv, wa_ref[...], preferred_element_type=jnp.float32) + ba_ref[...])
    or_ref[...] = jax.nn.relu(
        jnp.dot(xv, wb_ref[...], preferred_element_type=jnp.float32) + bb_ref[...])


def _proj(xp, Wpa, Wpb, bpa, bpb):
    return pl.pallas_call(
        _proj_body,
        grid=(NBLK,),
        in_specs=[
            pl.BlockSpec((BM, D_IN), lambda i: (i, 0)),
            pl.BlockSpec((D_IN, DHALF), lambda i: (0, 0)),
            pl.BlockSpec((D_IN, DHALF), lambda i: (0, 0)),
            pl.BlockSpec((1, DHALF), lambda i: (0, 0)),
            pl.BlockSpec((1, DHALF), lambda i: (0, 0)),
        ],
        out_specs=[pl.BlockSpec((BM, DHALF), lambda i: (i, 0)),
                   pl.BlockSpec((BM, DHALF), lambda i: (i, 0))],
        out_shape=[jax.ShapeDtypeStruct((NPAD, DHALF), jnp.float32),
                   jax.ShapeDtypeStruct((NPAD, DHALF), jnp.float32)],
    )(xp, Wpa, Wpb, bpa, bpb)


def _mlp_body(ul_ref, ur_ref, w1a_ref, w1b_ref, b1_ref,
              w2a_ref, w2b_ref, b2a_ref, b2b_ref, ol_ref, or_ref):
    ul = ul_ref[...]
    ur = ur_ref[...]
    u = jax.nn.relu(
        jnp.dot(ul, w1a_ref[...], preferred_element_type=jnp.float32)
        + jnp.dot(ur, w1b_ref[...], preferred_element_type=jnp.float32)
        + b1_ref[...])
    ol_ref[...] = jax.nn.relu(
        jnp.dot(u, w2a_ref[...], preferred_element_type=jnp.float32) + b2a_ref[...])
    or_ref[...] = jax.nn.relu(
        jnp.dot(u, w2b_ref[...], preferred_element_type=jnp.float32) + b2b_ref[...])


def _mlp(ul, ur, W1a, W1b, b1, W2a, W2b, b2a, b2b):
    return pl.pallas_call(
        _mlp_body,
        grid=(NBLK,),
        in_specs=[
            pl.BlockSpec((BM, DHALF), lambda i: (i, 0)),
            pl.BlockSpec((BM, DHALF), lambda i: (i, 0)),
            pl.BlockSpec((DHALF, DP), lambda i: (0, 0)),
            pl.BlockSpec((DHALF, DP), lambda i: (0, 0)),
            pl.BlockSpec((1, DP), lambda i: (0, 0)),
            pl.BlockSpec((DP, DHALF), lambda i: (0, 0)),
            pl.BlockSpec((DP, DHALF), lambda i: (0, 0)),
            pl.BlockSpec((1, DHALF), lambda i: (0, 0)),
            pl.BlockSpec((1, DHALF), lambda i: (0, 0)),
        ],
        out_specs=[pl.BlockSpec((BM, DHALF), lambda i: (i, 0)),
                   pl.BlockSpec((BM, DHALF), lambda i: (i, 0))],
        out_shape=[jax.ShapeDtypeStruct((NPAD, DHALF), jnp.float32),
                   jax.ShapeDtypeStruct((NPAD, DHALF), jnp.float32)],
    )(ul, ur, W1a, W1b, b1, W2a, W2b, b2a, b2b)


def _last_body(ul_ref, ur_ref, w1a_ref, w1b_ref, b1_ref,
               w2_ref, b2_ref, batch_ref, ws_ref, bs_ref, pw_ref,
               o_ref, acc_ref):
    i = pl.program_id(0)

    @pl.when(i == 0)
    def _():
        acc_ref[...] = jnp.zeros_like(acc_ref)

    ul = ul_ref[...]
    ur = ur_ref[...]
    u = jax.nn.relu(
        jnp.dot(ul, w1a_ref[...], preferred_element_type=jnp.float32)
        + jnp.dot(ur, w1b_ref[...], preferred_element_type=jnp.float32)
        + b1_ref[...])
    t = jnp.dot(u, w2_ref[...], preferred_element_type=jnp.float32) + b2_ref[...]
    gids = batch_ref[...]
    seg = jax.lax.broadcasted_iota(jnp.int32, (NG, BM), 0)
    onehot = (seg == gids).astype(jnp.float32)
    acc_ref[...] += jnp.dot(onehot, t, preferred_element_type=jnp.float32)

    @pl.when(i == NBLK - 1)
    def _():
        r = jnp.dot(acc_ref[...], ws_ref[...],
                    preferred_element_type=jnp.float32) + bs_ref[...]
        pw = pw_ref[0]
        o_ref[...] = jnp.where(r >= 0.0, r, pw * r)


def _last(ul, ur, W1a, W1b, b1, W2f, b2f, batch2d, Wsp, bs2d, pw):
    return pl.pallas_call(
        _last_body,
        grid=(NBLK,),
        in_specs=[
            pl.BlockSpec((BM, DHALF), lambda i: (i, 0)),
            pl.BlockSpec((BM, DHALF), lambda i: (i, 0)),
            pl.BlockSpec((DHALF, DP), lambda i: (0, 0)),
            pl.BlockSpec((DHALF, DP), lambda i: (0, 0)),
            pl.BlockSpec((1, DP), lambda i: (0, 0)),
            pl.BlockSpec((DP, DP), lambda i: (0, 0)),
            pl.BlockSpec((1, DP), lambda i: (0, 0)),
            pl.BlockSpec((1, BM), lambda i: (0, i)),
            pl.BlockSpec((DP, DOUT), lambda i: (0, 0)),
            pl.BlockSpec((1, DOUT), lambda i: (0, 0)),
            pl.BlockSpec(memory_space=pltpu.SMEM),
        ],
        out_specs=pl.BlockSpec((NG, DOUT), lambda i: (0, 0)),
        out_shape=jax.ShapeDtypeStruct((NG, DOUT), jnp.float32),
        scratch_shapes=[pltpu.VMEM((NG, DP), jnp.float32)],
    )(ul, ur, W1a, W1b, b1, W2f, b2f, batch2d, Wsp, bs2d, pw)


def kernel(x, edge_index, batch, Wp, bp, W1, b1, W2, b2, Ws, bs, prelu_w):
    src = edge_index[0]
    dst = edge_index[1]
    xp = jnp.pad(x, ((0, NPAD - N), (0, 0)))
    Wpp = jnp.pad(Wp, ((0, 0), (0, DP - DH)))
    bpp = jnp.pad(bp, ((0, DP - DH),)).reshape(1, DP)
    W1p = jnp.pad(W1, ((0, 0), (0, DP - DH), (0, DP - DH)))
    b1p = jnp.pad(b1, ((0, 0), (0, DP - DH))).reshape(DEPTH, 1, DP)
    W2p = jnp.pad(W2, ((0, 0), (0, DP - DH), (0, DP - DH)))
    b2p = jnp.pad(b2, ((0, 0), (0, DP - DH))).reshape(DEPTH, 1, DP)
    Wsp = jnp.pad(Ws, ((0, DP - DH), (0, 0)))
    bs2d = bs.reshape(1, DOUT)
    batch2d = jnp.pad(batch, (0, NPAD - N), constant_values=NG).reshape(1, NPAD)
    pw = prelu_w.reshape(1)

    Wpa, Wpb = Wpp[:, :DHALF], Wpp[:, DHALF:]
    bpa, bpb = bpp[:, :DHALF], bpp[:, DHALF:]
    W1a, W1b = W1p[:, :DHALF, :], W1p[:, DHALF:, :]
    W2a, W2b = W2p[:, :, :DHALF], W2p[:, :, DHALF:]
    b2a, b2b = b2p[:, :, :DHALF], b2p[:, :, DHALF:]

    hl, hr = _proj(xp, Wpa, Wpb, bpa, bpb)
    for i in range(DEPTH):
        ul, ur = _sc_agg(hl, hr, src, dst)
        if i < DEPTH - 1:
            hl, hr = _mlp(ul, ur, W1a[i], W1b[i], b1p[i],
                          W2a[i], W2b[i], b2a[i], b2b[i])
        else:
            r = _last(ul, ur, W1a[i], W1b[i], b1p[i],
                      W2p[i], b2p[i], batch2d, Wsp, bs2d, pw)
    return r


# speedup vs baseline: 6.8813x; 1.0244x over previous
import functools

import jax
import jax.numpy as jnp
from jax import lax
from jax.experimental import pallas as pl
from jax.experimental.pallas import tpu as pltpu
from jax.experimental.pallas import tpu_sc as plsc

N = 10000
E = 320000
D_IN = 128
DH = 300
DP = 320
DHALF = DP // 2
DOUT = 1024
DEPTH = 5
NG = 128
NPAD = 10240
BM = 1024
NBLK = NPAD // BM

NS = 16
EPT = E // NS
CH = 80
NCHUNK = EPT // CH
RPT = NPAD // NS

_sc_mesh = plsc.VectorSubcoreMesh(core_axis_name="c", subcore_axis_name="s")


@functools.partial(
    pl.kernel,
    out_type=(jax.ShapeDtypeStruct((NPAD, DHALF), jnp.float32),
              jax.ShapeDtypeStruct((NPAD, DHALF), jnp.float32)),
    mesh=_sc_mesh,
    scratch_types=[
        pltpu.VMEM_SHARED((NPAD, DHALF), jnp.float32),
        pltpu.VMEM((CH, DHALF), jnp.float32),
        pltpu.VMEM((CH, DHALF), jnp.float32),
        pltpu.VMEM((CH,), jnp.int32),
        pltpu.VMEM((CH,), jnp.int32),
        pltpu.VMEM((CH,), jnp.int32),
        pltpu.VMEM((CH,), jnp.int32),
        pltpu.VMEM((CH,), jnp.int32),
        pltpu.VMEM((CH,), jnp.int32),
        pltpu.VMEM((CH,), jnp.int32),
        pltpu.VMEM((CH,), jnp.int32),
        pltpu.SemaphoreType.DMA,
        pltpu.SemaphoreType.DMA,
        pltpu.SemaphoreType.DMA,
        pltpu.SemaphoreType.DMA,
        pltpu.SemaphoreType.DMA,
        pltpu.SemaphoreType.DMA,
        pltpu.SemaphoreType.DMA,
        pltpu.SemaphoreType.DMA,
        pltpu.SemaphoreType.DMA,
        pltpu.SemaphoreType.DMA,
        pltpu.SemaphoreType.DMA,
        pltpu.SemaphoreType.DMA,
    ],
    compiler_params=pltpu.CompilerParams(use_tc_tiling_on_sc=False),
)
def _sc_agg(hl_hbm, hr_hbm, src_hbm, dst_hbm,
            outl_hbm, outr_hbm,
            acc_sh, rows0, rows1, sb0, sb1, sb2, sb3, db0, db1, db2, db3,
            sg0, sg1, ss0, ss1, sia0, sia1, sia2, sia3,
            sib0, sib1, sib2, sib3):
    c = lax.axis_index("c")
    s = lax.axis_index("s")
    rows = [rows0, rows1]
    srcb = [sb0, sb1, sb2, sb3]
    dstb = [db0, db1, db2, db3]
    sem_g = [sg0, sg1]
    sem_s = [ss0, ss1]
    sem_ia = [sia0, sia1, sia2, sia3]
    sem_ib = [sib0, sib1, sib2, sib3]

    r0 = s * RPT

    @pl.when(c == 0)
    def _():
        pltpu.sync_copy(hl_hbm.at[pl.ds(r0, RPT), :],
                        acc_sh.at[pl.ds(r0, RPT), :])

    @pl.when(c == 1)
    def _():
        pltpu.sync_copy(hr_hbm.at[pl.ds(r0, RPT), :],
                        acc_sh.at[pl.ds(r0, RPT), :])

    plsc.subcore_barrier()

    base = s * EPT

    def do_edges(h_hbm):
        def idx_start(ci, st):
            off = pl.multiple_of(base + ci * CH, 8)
            pltpu.async_copy(src_hbm.at[pl.ds(off, CH)], srcb[st], sem_ia[st])
            pltpu.async_copy(dst_hbm.at[pl.ds(off, CH)], dstb[st], sem_ib[st])

        def idx_wait(ci, st):
            off = pl.multiple_of(base + ci * CH, 8)
            pltpu.make_async_copy(src_hbm.at[pl.ds(off, CH)], srcb[st],
                                  sem_ia[st]).wait()
            pltpu.make_async_copy(dst_hbm.at[pl.ds(off, CH)], dstb[st],
                                  sem_ib[st]).wait()

        def gather_start(st, rb):
            pltpu.async_copy(h_hbm.at[srcb[st]], rows[rb], sem_g[rb])

        def gather_wait(st, rb):
            pltpu.make_async_copy(h_hbm.at[srcb[st]], rows[rb],
                                  sem_g[rb]).wait()

        def scat_start(st, rb):
            pltpu.async_copy(rows[rb], acc_sh.at[dstb[st]], sem_s[rb],
                             add=True)

        def scat_wait(st, rb):
            pltpu.make_async_copy(rows[rb], acc_sh.at[dstb[st]],
                                  sem_s[rb]).wait()

        for j in range(4):
            idx_start(j, j)
        idx_wait(0, 0)
        gather_start(0, 0)
        idx_wait(1, 1)
        gather_start(1, 1)
        gather_wait(0, 0)
        scat_start(0, 0)

        def body(k, carry):
            for b in range(4):
                i = 2 + 4 * k + b
                rb = b & 1
                st = (2 + b) & 3
                rb_prev = (1 + b) & 1
                st_prev = (1 + b) & 3
                st_old = b & 3
                scat_wait(st_old, rb)
                idx_wait(i, st)
                gather_start(st, rb)
                gather_wait(st_prev, rb_prev)
                scat_start(st_prev, rb_prev)
                if b < 2:
                    idx_start(i + 2, st_old)
                else:
                    @pl.when(k < NCHUNK // 4 - 1)
                    def _():
                        idx_start(i + 2, st_old)
            return carry
        lax.fori_loop(0, (NCHUNK - 2) // 4, body, 0)

        last = NCHUNK - 1
        gather_wait(last & 3, last & 1)
        scat_start(last & 3, last & 1)
        scat_wait((last - 1) & 3, (last - 1) & 1)
        scat_wait(last & 3, last & 1)

    @pl.when(c == 0)
    def _():
        do_edges(hl_hbm)

    @pl.when(c == 1)
    def _():
        do_edges(hr_hbm)

    plsc.subcore_barrier()

    def dump(out_hbm):
        pltpu.sync_copy(acc_sh.at[pl.ds(r0, RPT), :],
                        out_hbm.at[pl.ds(r0, RPT), :])

    @pl.when(c == 0)
    def _():
        dump(outl_hbm)

    @pl.when(c == 1)
    def _():
        dump(outr_hbm)


def _proj_body(x_ref, wa_ref, wb_ref, ba_ref, bb_ref, ol_ref, or_ref):
    xv = x_ref[...]
    ol_ref[...] = jax.nn.relu(
        jnp.dot(xv, wa_ref[...], preferred_element_type=jnp.float32) + ba_ref[...])
    or_ref[...] = jax.nn.relu(
        jnp.dot(xv, wb_ref[...], preferred_element_type=jnp.float32) + bb_ref[...])


def _proj(xp, Wpa, Wpb, bpa, bpb):
    return pl.pallas_call(
        _proj_body,
        grid=(NBLK,),
        in_specs=[
            pl.BlockSpec((BM, D_IN), lambda i: (i, 0)),
            pl.BlockSpec((D_IN, DHALF), lambda i: (0, 0)),
            pl.BlockSpec((D_IN, DHALF), lambda i: (0, 0)),
            pl.BlockSpec((1, DHALF), lambda i: (0, 0)),
            pl.BlockSpec((1, DHALF), lambda i: (0, 0)),
        ],
        out_specs=[pl.BlockSpec((BM, DHALF), lambda i: (i, 0)),
                   pl.BlockSpec((BM, DHALF), lambda i: (i, 0))],
        out_shape=[jax.ShapeDtypeStruct((NPAD, DHALF), jnp.float32),
                   jax.ShapeDtypeStruct((NPAD, DHALF), jnp.float32)],
    )(xp, Wpa, Wpb, bpa, bpb)


def _mlp_body(ul_ref, ur_ref, w1a_ref, w1b_ref, b1_ref,
              w2a_ref, w2b_ref, b2a_ref, b2b_ref, ol_ref, or_ref):
    ul = ul_ref[...]
    ur = ur_ref[...]
    u = jax.nn.relu(
        jnp.dot(ul, w1a_ref[...], preferred_element_type=jnp.float32)
        + jnp.dot(ur, w1b_ref[...], preferred_element_type=jnp.float32)
        + b1_ref[...])
    ol_ref[...] = jax.nn.relu(
        jnp.dot(u, w2a_ref[...], preferred_element_type=jnp.float32) + b2a_ref[...])
    or_ref[...] = jax.nn.relu(
        jnp.dot(u, w2b_ref[...], preferred_element_type=jnp.float32) + b2b_ref[...])


def _mlp(ul, ur, W1a, W1b, b1, W2a, W2b, b2a, b2b):
    return pl.pallas_call(
        _mlp_body,
        grid=(NBLK,),
        in_specs=[
            pl.BlockSpec((BM, DHALF), lambda i: (i, 0)),
            pl.BlockSpec((BM, DHALF), lambda i: (i, 0)),
            pl.BlockSpec((DHALF, DP), lambda i: (0, 0)),
            pl.BlockSpec((DHALF, DP), lambda i: (0, 0)),
            pl.BlockSpec((1, DP), lambda i: (0, 0)),
            pl.BlockSpec((DP, DHALF), lambda i: (0, 0)),
            pl.BlockSpec((DP, DHALF), lambda i: (0, 0)),
            pl.BlockSpec((1, DHALF), lambda i: (0, 0)),
            pl.BlockSpec((1, DHALF), lambda i: (0, 0)),
        ],
        out_specs=[pl.BlockSpec((BM, DHALF), lambda i: (i, 0)),
                   pl.BlockSpec((BM, DHALF), lambda i: (i, 0))],
        out_shape=[jax.ShapeDtypeStruct((NPAD, DHALF), jnp.float32),
                   jax.ShapeDtypeStruct((NPAD, DHALF), jnp.float32)],
    )(ul, ur, W1a, W1b, b1, W2a, W2b, b2a, b2b)


def _last_body(ul_ref, ur_ref, w1a_ref, w1b_ref, b1_ref,
               w2_ref, b2_ref, batch_ref, ws_ref, bs_ref, pw_ref,
               o_ref, acc_ref):
    i = pl.program_id(0)

    @pl.when(i == 0)
    def _():
        acc_ref[...] = jnp.zeros_like(acc_ref)

    ul = ul_ref[...]
    ur = ur_ref[...]
    u = jax.nn.relu(
        jnp.dot(ul, w1a_ref[...], preferred_element_type=jnp.float32)
        + jnp.dot(ur, w1b_ref[...], preferred_element_type=jnp.float32)
        + b1_ref[...])
    t = jnp.dot(u, w2_ref[...], preferred_element_type=jnp.float32) + b2_ref[...]
    gids = batch_ref[...]
    seg = jax.lax.broadcasted_iota(jnp.int32, (NG, BM), 0)
    onehot = (seg == gids).astype(jnp.float32)
    acc_ref[...] += jnp.dot(onehot, t, preferred_element_type=jnp.float32)

    @pl.when(i == NBLK - 1)
    def _():
        r = jnp.dot(acc_ref[...], ws_ref[...],
                    preferred_element_type=jnp.float32) + bs_ref[...]
        pw = pw_ref[0]
        o_ref[...] = jnp.where(r >= 0.0, r, pw * r)


def _last(ul, ur, W1a, W1b, b1, W2f, b2f, batch2d, Wsp, bs2d, pw):
    return pl.pallas_call(
        _last_body,
        grid=(NBLK,),
        in_specs=[
            pl.BlockSpec((BM, DHALF), lambda i: (i, 0)),
            pl.BlockSpec((BM, DHALF), lambda i: (i, 0)),
            pl.BlockSpec((DHALF, DP), lambda i: (0, 0)),
            pl.BlockSpec((DHALF, DP), lambda i: (0, 0)),
            pl.BlockSpec((1, DP), lambda i: (0, 0)),
            pl.BlockSpec((DP, DP), lambda i: (0, 0)),
            pl.BlockSpec((1, DP), lambda i: (0, 0)),
            pl.BlockSpec((1, BM), lambda i: (0, i)),
            pl.BlockSpec((DP, DOUT), lambda i: (0, 0)),
            pl.BlockSpec((1, DOUT), lambda i: (0, 0)),
            pl.BlockSpec(memory_space=pltpu.SMEM),
        ],
        out_specs=pl.BlockSpec((NG, DOUT), lambda i: (0, 0)),
        out_shape=jax.ShapeDtypeStruct((NG, DOUT), jnp.float32),
        scratch_shapes=[pltpu.VMEM((NG, DP), jnp.float32)],
    )(ul, ur, W1a, W1b, b1, W2f, b2f, batch2d, Wsp, bs2d, pw)


def kernel(x, edge_index, batch, Wp, bp, W1, b1, W2, b2, Ws, bs, prelu_w):
    src = edge_index[0]
    dst = edge_index[1]
    xp = jnp.pad(x, ((0, NPAD - N), (0, 0)))
    Wpp = jnp.pad(Wp, ((0, 0), (0, DP - DH)))
    bpp = jnp.pad(bp, ((0, DP - DH),)).reshape(1, DP)
    W1p = jnp.pad(W1, ((0, 0), (0, DP - DH), (0, DP - DH)))
    b1p = jnp.pad(b1, ((0, 0), (0, DP - DH))).reshape(DEPTH, 1, DP)
    W2p = jnp.pad(W2, ((0, 0), (0, DP - DH), (0, DP - DH)))
    b2p = jnp.pad(b2, ((0, 0), (0, DP - DH))).reshape(DEPTH, 1, DP)
    Wsp = jnp.pad(Ws, ((0, DP - DH), (0, 0)))
    bs2d = bs.reshape(1, DOUT)
    batch2d = jnp.pad(batch, (0, NPAD - N), constant_values=NG).reshape(1, NPAD)
    pw = prelu_w.reshape(1)

    Wpa, Wpb = Wpp[:, :DHALF], Wpp[:, DHALF:]
    bpa, bpb = bpp[:, :DHALF], bpp[:, DHALF:]
    W1a, W1b = W1p[:, :DHALF, :], W1p[:, DHALF:, :]
    W2a, W2b = W2p[:, :, :DHALF], W2p[:, :, DHALF:]
    b2a, b2b = b2p[:, :, :DHALF], b2p[:, :, DHALF:]

    hl, hr = _proj(xp, Wpa, Wpb, bpa, bpb)
    for i in range(DEPTH):
        ul, ur = _sc_agg(hl, hr, src, dst)
        if i < DEPTH - 1:
            hl, hr = _mlp(ul, ur, W1a[i], W1b[i], b1p[i],
                          W2a[i], W2b[i], b2a[i], b2b[i])
        else:
            r = _last(ul, ur, W1a[i], W1b[i], b1p[i],
                      W2p[i], b2p[i], batch2d, Wsp, bs2d, pw)
    return r
